# baseline (speedup 1.0000x reference)
import functools

import jax
import jax.numpy as jnp
from jax import lax
from jax.experimental import pallas as pl
from jax.experimental.pallas import tpu as pltpu
from jax.experimental.pallas import tpu_sc as plsc

Q = 2048
N = 65536
D = 32
K = 8
TILE = 1024
NTILES = N // TILE
NCAND = NTILES * K
LANES = 16
NWORKERS = 32
QPW = Q // NWORKERS
RPW = QPW * K
IDX_CHUNK = 128
NCHUNKS = RPW // IDX_CHUNK


def _vi_tree(v, g, payloads=()):
    r = v.shape[0]
    while r > 1:
        h2 = r // 2
        v1, v2 = v[0:h2], v[h2:r]
        g1, g2 = g[0:h2], g[h2:r]
        s = (v2 < v1) | ((v2 == v1) & (g2 < g1))
        v = jnp.where(s, v2, v1)
        g = jnp.where(s, g2, g1)
        payloads = tuple(jnp.where(s, a[h2:r], a[0:h2]) for a in payloads)
        r = h2
    return (v, g) + payloads


def _topk_tc_kernel(x_ref, qt_ref, w_out, ix_out, cand_v, cand_i):
    t = pl.program_id(0)

    xb = x_ref[...]
    qt = qt_ref[...]
    xsq = jnp.sum(xb * xb, axis=1, keepdims=True)
    qsq = jnp.sum(qt * qt, axis=0, keepdims=True)
    mm = lax.dot_general(xb, qt, (((1,), (0,)), ((), ())),
                         preferred_element_type=jnp.float32,
                         precision=lax.Precision.DEFAULT)
    d2 = (qsq - 2.0 * mm) + xsq

    h = TILE // 2
    va = d2[0:h, :]
    vb = d2[h:TILE, :]
    s = vb < va
    pv = jnp.where(s, vb, va)
    rv = jnp.where(s, va, vb)
    iota_h = lax.broadcasted_iota(jnp.int32, (h, Q), 0).astype(jnp.float32)
    half = jnp.where(s, float(h), 0.0)
    gv = iota_h + half
    g2 = iota_h + (float(h) - half)
    tile_v = []
    tile_i = []
    for _ in range(K):
        m, p = _vi_tree(pv, gv)
        tile_v.append(m)
        tile_i.append(p.astype(jnp.int32))
        sel = gv == p
        pv = jnp.where(sel, rv, pv)
        gv = jnp.where(sel, g2, gv)
        rv = jnp.where(sel, jnp.inf, rv)

    cand_v[pl.ds(t * K, K), :] = jnp.concatenate(tile_v, axis=0)
    cand_i[pl.ds(t * K, K), :] = (jnp.concatenate(tile_i, axis=0)
                                  + t * TILE)

    @pl.when(t == NTILES - 1)
    def _epilogue():
        cv = cand_v[...]
        ci = cand_i[...]
        iota2_f = lax.broadcasted_iota(jnp.int32, (NCAND, Q), 0).astype(jnp.float32)
        best_v = []
        best_i = []
        for _ in range(K):
            m, p, gi = _vi_tree(cv, iota2_f, (ci,))
            best_v.append(m)
            best_i.append(gi)
            cv = jnp.where(iota2_f == p, jnp.inf, cv)
        d2b = jnp.concatenate(best_v, axis=0)
        dist = jnp.sqrt(jnp.maximum(d2b, 1e-12))
        sigma_sq = jnp.square(jnp.max(dist)) / 9.0
        w = jnp.exp(-jnp.square(dist) / (2.0 * sigma_sq))
        w = w / jnp.sum(w, axis=0, keepdims=True)
        w_out[...] = w
        ix_out[...] = jnp.concatenate(best_i, axis=0)


def _topk_weights(x, qt):
    return pl.pallas_call(
        _topk_tc_kernel,
        grid=(NTILES,),
        in_specs=[
            pl.BlockSpec((TILE, D), lambda t: (t, 0)),
            pl.BlockSpec((D, Q), lambda t: (0, 0)),
        ],
        out_specs=[
            pl.BlockSpec((K, Q), lambda t: (0, 0)),
            pl.BlockSpec((K, Q), lambda t: (0, 0)),
        ],
        out_shape=[
            jax.ShapeDtypeStruct((K, Q), jnp.float32),
            jax.ShapeDtypeStruct((K, Q), jnp.int32),
        ],
        scratch_shapes=[
            pltpu.VMEM((NCAND, Q), jnp.float32),
            pltpu.VMEM((NCAND, Q), jnp.int32),
        ],
    )(x, qt)


def _gather_sc_kernel(zt_hbm, idx_hbm, w_hbm, out_hbm,
                      idx_v, rows_v, w_v, out_v, sem):
    nc = plsc.get_sparse_core_info().num_cores
    wid = lax.axis_index("s") * nc + lax.axis_index("c")
    base = wid * RPW

    pltpu.sync_copy(idx_hbm.at[wid], idx_v)
    copies = []
    for c in range(NCHUNKS):
        copies.append(pltpu.async_copy(
            zt_hbm.at[idx_v.at[c]],
            rows_v.at[pl.ds(c * IDX_CHUNK, IDX_CHUNK)], sem))
    pltpu.sync_copy(w_hbm.at[pl.ds(base, RPW)], w_v)
    for cp in copies:
        cp.wait()

    def body(qi, _):
        r0 = qi * K
        acc0 = jnp.zeros((LANES,), jnp.float32)
        acc1 = jnp.zeros((LANES,), jnp.float32)
        for j in range(K):
            wv = w_v[r0 + j, :]
            acc0 = acc0 + rows_v[r0 + j, pl.ds(0, LANES)] * wv
            acc1 = acc1 + rows_v[r0 + j, pl.ds(LANES, LANES)] * wv
        out_v[qi, pl.ds(0, LANES)] = acc0
        out_v[qi, pl.ds(LANES, LANES)] = acc1
        return 0

    lax.fori_loop(0, QPW, body, 0)
    pltpu.sync_copy(out_v, out_hbm.at[pl.ds(wid * QPW, QPW)])


@functools.cache
def _weighted_gather():
    @functools.partial(
        pl.kernel,
        out_type=jax.ShapeDtypeStruct((Q, D), jnp.float32),
        mesh=plsc.VectorSubcoreMesh(core_axis_name="c", subcore_axis_name="s"),
        compiler_params=pltpu.CompilerParams(use_tc_tiling_on_sc=False),
        scratch_types=[
            pltpu.VMEM((NCHUNKS, IDX_CHUNK), jnp.int32),
            pltpu.VMEM((RPW, D), jnp.float32),
            pltpu.VMEM((RPW, LANES), jnp.float32),
            pltpu.VMEM((QPW, D), jnp.float32),
            pltpu.SemaphoreType.DMA,
        ],
    )
    def run(zt, idx, w, out, *scratch):
        _gather_sc_kernel(zt, idx, w, out, *scratch)

    return run


def kernel(x, q, z):
    w_kq, ix_kq = _topk_weights(x, q.T)
    zt = z.T
    idx = ix_kq.T.reshape(NWORKERS, NCHUNKS, IDX_CHUNK)
    w2 = jnp.broadcast_to(w_kq.T.reshape(Q * K, 1), (Q * K, LANES))
    out = _weighted_gather()(zt, idx, w2)
    return out.T


# speedup vs baseline: 2.6549x; 2.6549x over previous
import functools

import jax
import jax.numpy as jnp
from jax import lax
from jax.experimental import pallas as pl
from jax.experimental.pallas import tpu as pltpu
from jax.experimental.pallas import tpu_sc as plsc

Q = 2048
N = 65536
D = 32
K = 8
TILE = 1024
NTILES = N // TILE
NCAND = NTILES * K
LANES = 16
NWORKERS = 32
QPW = Q // NWORKERS
RPW = QPW * K
IDX_CHUNK = 128
NCHUNKS = RPW // IDX_CHUNK


def _topk_tc_kernel(x_ref, qt_ref, w_out, ix_out, cand_v, cand_i):
    t = pl.program_id(0)

    xb = x_ref[...]
    qt = qt_ref[...]
    xsq = jnp.sum(xb * xb, axis=1, keepdims=True)
    qsq = jnp.sum(qt * qt, axis=0, keepdims=True)
    mm = lax.dot_general(xb, qt, (((1,), (0,)), ((), ())),
                         preferred_element_type=jnp.float32,
                         precision=lax.Precision.DEFAULT)
    d2 = (qsq - 2.0 * mm) + xsq

    h = TILE // 2
    va = d2[0:h, :]
    vb = d2[h:TILE, :]
    s = vb < va
    pv = jnp.where(s, vb, va)
    rv = jnp.where(s, va, vb)
    iota_h = lax.broadcasted_iota(jnp.int32, (h, Q), 0).astype(jnp.float32)
    half = jnp.where(s, float(h), 0.0)
    gv = iota_h + half
    g2 = iota_h + (float(h) - half)
    tile_v = []
    tile_i = []
    for _ in range(K):
        m = jnp.min(pv, axis=0, keepdims=True)
        y = jnp.where(pv == m, gv, float(TILE))
        p = jnp.min(y, axis=0, keepdims=True)
        tile_v.append(m)
        tile_i.append(p.astype(jnp.int32))
        sel = gv == p
        pv = jnp.where(sel, rv, pv)
        gv = jnp.where(sel, g2, gv)
        rv = jnp.where(sel, jnp.inf, rv)

    cand_v[pl.ds(t * K, K), :] = jnp.concatenate(tile_v, axis=0)
    cand_i[pl.ds(t * K, K), :] = (jnp.concatenate(tile_i, axis=0)
                                  + t * TILE)

    @pl.when(t == NTILES - 1)
    def _epilogue():
        cv = cand_v[...]
        ci = cand_i[...]
        iota2_f = lax.broadcasted_iota(jnp.int32, (NCAND, Q), 0).astype(jnp.float32)
        best_v = []
        best_i = []
        for _ in range(K):
            m = jnp.min(cv, axis=0, keepdims=True)
            y = jnp.where(cv == m, iota2_f, float(NCAND))
            p = jnp.min(y, axis=0, keepdims=True)
            sel = iota2_f == p
            gi = jnp.sum(jnp.where(sel, ci, 0), axis=0, keepdims=True)
            best_v.append(m)
            best_i.append(gi)
            cv = jnp.where(sel, jnp.inf, cv)
        d2b = jnp.concatenate(best_v, axis=0)
        dist = jnp.sqrt(jnp.maximum(d2b, 1e-12))
        sigma_sq = jnp.square(jnp.max(dist)) / 9.0
        w = jnp.exp(-jnp.square(dist) / (2.0 * sigma_sq))
        w = w / jnp.sum(w, axis=0, keepdims=True)
        w_out[...] = w
        ix_out[...] = jnp.concatenate(best_i, axis=0)


def _topk_weights(x, qt):
    return pl.pallas_call(
        _topk_tc_kernel,
        grid=(NTILES,),
        in_specs=[
            pl.BlockSpec((TILE, D), lambda t: (t, 0)),
            pl.BlockSpec((D, Q), lambda t: (0, 0)),
        ],
        out_specs=[
            pl.BlockSpec((K, Q), lambda t: (0, 0)),
            pl.BlockSpec((K, Q), lambda t: (0, 0)),
        ],
        out_shape=[
            jax.ShapeDtypeStruct((K, Q), jnp.float32),
            jax.ShapeDtypeStruct((K, Q), jnp.int32),
        ],
        scratch_shapes=[
            pltpu.VMEM((NCAND, Q), jnp.float32),
            pltpu.VMEM((NCAND, Q), jnp.int32),
        ],
    )(x, qt)


def _gather_sc_kernel(zt_hbm, idx_hbm, w_hbm, out_hbm,
                      idx_v, rows_v, w_v, out_v, sem):
    nc = plsc.get_sparse_core_info().num_cores
    wid = lax.axis_index("s") * nc + lax.axis_index("c")
    base = wid * RPW

    pltpu.sync_copy(idx_hbm.at[wid], idx_v)
    copies = []
    for c in range(NCHUNKS):
        copies.append(pltpu.async_copy(
            zt_hbm.at[idx_v.at[c]],
            rows_v.at[pl.ds(c * IDX_CHUNK, IDX_CHUNK)], sem))
    pltpu.sync_copy(w_hbm.at[pl.ds(base, RPW)], w_v)
    for cp in copies:
        cp.wait()

    def body(qi, _):
        r0 = qi * K
        acc0 = jnp.zeros((LANES,), jnp.float32)
        acc1 = jnp.zeros((LANES,), jnp.float32)
        for j in range(K):
            wv = w_v[r0 + j, :]
            acc0 = acc0 + rows_v[r0 + j, pl.ds(0, LANES)] * wv
            acc1 = acc1 + rows_v[r0 + j, pl.ds(LANES, LANES)] * wv
        out_v[qi, pl.ds(0, LANES)] = acc0
        out_v[qi, pl.ds(LANES, LANES)] = acc1
        return 0

    lax.fori_loop(0, QPW, body, 0)
    pltpu.sync_copy(out_v, out_hbm.at[pl.ds(wid * QPW, QPW)])


@functools.cache
def _weighted_gather():
    @functools.partial(
        pl.kernel,
        out_type=jax.ShapeDtypeStruct((Q, D), jnp.float32),
        mesh=plsc.VectorSubcoreMesh(core_axis_name="c", subcore_axis_name="s"),
        compiler_params=pltpu.CompilerParams(use_tc_tiling_on_sc=False),
        scratch_types=[
            pltpu.VMEM((NCHUNKS, IDX_CHUNK), jnp.int32),
            pltpu.VMEM((RPW, D), jnp.float32),
            pltpu.VMEM((RPW, LANES), jnp.float32),
            pltpu.VMEM((QPW, D), jnp.float32),
            pltpu.SemaphoreType.DMA,
        ],
    )
    def run(zt, idx, w, out, *scratch):
        _gather_sc_kernel(zt, idx, w, out, *scratch)

    return run


def kernel(x, q, z):
    w_kq, ix_kq = _topk_weights(x, q.T)
    zt = z.T
    idx = ix_kq.T.reshape(NWORKERS, NCHUNKS, IDX_CHUNK)
    w2 = jnp.broadcast_to(w_kq.T.reshape(Q * K, 1), (Q * K, LANES))
    out = _weighted_gather()(zt, idx, w2)
    return out.T
